# baseline (speedup 1.0000x reference)
import jax
import jax.numpy as jnp
from jax.experimental import pallas as pl
import jax.experimental.pallas.tpu as pltpu

_N = 10000
_D = 128
_K = 4
_DS = 32
_TILE = 1000


def _fused_body(x_ref, w_ref, b_ref, a_ref, z_ref, z2_ref):
    y = jnp.dot(x_ref[...], w_ref[...], preferred_element_type=jnp.float32)
    y = y + b_ref[...]
    y = jnp.maximum(y, 0.0)
    s = jnp.dot(y * y, a_ref[...], preferred_element_type=jnp.float32)
    out = y / jnp.maximum(jnp.sqrt(s), 1e-12)
    z_ref[...] = out
    z2_ref[...] = out


def kernel(X, edges, W, b):
    del edges
    W2 = jnp.transpose(W, (1, 0, 2)).reshape(_D, _K * _DS)
    b2 = b.reshape(1, _K * _DS)
    A = jnp.tile(jnp.eye(_DS, dtype=jnp.float32), (_K, _K))
    grid = (_N // _TILE,)
    z, z2 = pl.pallas_call(
        _fused_body,
        grid=grid,
        in_specs=[
            pl.BlockSpec((_TILE, _D), lambda i: (i, 0)),
            pl.BlockSpec((_D, _K * _DS), lambda i: (0, 0)),
            pl.BlockSpec((1, _K * _DS), lambda i: (0, 0)),
            pl.BlockSpec((_K * _DS, _K * _DS), lambda i: (0, 0)),
        ],
        out_specs=[
            pl.BlockSpec((_TILE, _K * _DS), lambda i: (i, 0)),
            pl.BlockSpec((_TILE, _K * _DS), lambda i: (i, 0)),
        ],
        out_shape=[
            jax.ShapeDtypeStruct((_N, _K * _DS), jnp.float32),
            jax.ShapeDtypeStruct((_N, _K * _DS), jnp.float32),
        ],
        compiler_params=pltpu.CompilerParams(
            dimension_semantics=("parallel",),
        ),
    )(X, W2, b2, A)
    Z = z.reshape(_N, _K, _DS)
    _Z = z2.reshape(_N, 1, _K, _DS)
    return (Z, _Z)


# speedup vs baseline: 1.1055x; 1.1055x over previous
import jax
import jax.numpy as jnp
from jax.experimental import pallas as pl
import jax.experimental.pallas.tpu as pltpu

_N = 10000
_D = 128
_K = 4
_DS = 32
_TILE = 2000


def _fused_body(x_ref, w_ref, b_ref, a_ref, z_ref, z2_ref):
    y = jnp.dot(x_ref[...], w_ref[...], preferred_element_type=jnp.float32)
    y = y + b_ref[...]
    y = jnp.maximum(y, 0.0)
    s = jnp.dot(y * y, a_ref[...], preferred_element_type=jnp.float32)
    out = y / jnp.maximum(jnp.sqrt(s), 1e-12)
    z_ref[...] = out
    z2_ref[...] = out


def kernel(X, edges, W, b):
    del edges
    W2 = jnp.transpose(W, (1, 0, 2)).reshape(_D, _K * _DS)
    b2 = b.reshape(1, _K * _DS)
    A = jnp.tile(jnp.eye(_DS, dtype=jnp.float32), (_K, _K))
    grid = (_N // _TILE,)
    z, z2 = pl.pallas_call(
        _fused_body,
        grid=grid,
        in_specs=[
            pl.BlockSpec((_TILE, _D), lambda i: (i, 0)),
            pl.BlockSpec((_D, _K * _DS), lambda i: (0, 0)),
            pl.BlockSpec((1, _K * _DS), lambda i: (0, 0)),
            pl.BlockSpec((_K * _DS, _K * _DS), lambda i: (0, 0)),
        ],
        out_specs=[
            pl.BlockSpec((_TILE, _K * _DS), lambda i: (i, 0)),
            pl.BlockSpec((_TILE, _K * _DS), lambda i: (i, 0)),
        ],
        out_shape=[
            jax.ShapeDtypeStruct((_N, _K * _DS), jnp.float32),
            jax.ShapeDtypeStruct((_N, _K * _DS), jnp.float32),
        ],
        compiler_params=pltpu.CompilerParams(
            dimension_semantics=("parallel",),
        ),
    )(X, W2, b2, A)
    Z = z.reshape(_N, _K, _DS)
    _Z = z2.reshape(_N, 1, _K, _DS)
    return (Z, _Z)


# speedup vs baseline: 1.1902x; 1.0767x over previous
import jax
import jax.numpy as jnp
from jax.experimental import pallas as pl
import jax.experimental.pallas.tpu as pltpu

_N = 10000
_D = 128
_K = 4
_DS = 32
_TILE = 5000


def _fused_body(x_ref, w_ref, b_ref, a_ref, z_ref, z2_ref):
    y = jnp.dot(x_ref[...], w_ref[...], preferred_element_type=jnp.float32)
    y = y + b_ref[...]
    y = jnp.maximum(y, 0.0)
    s = jnp.dot(y * y, a_ref[...], preferred_element_type=jnp.float32)
    out = y / jnp.maximum(jnp.sqrt(s), 1e-12)
    z_ref[...] = out
    z2_ref[...] = out


def kernel(X, edges, W, b):
    del edges
    W2 = jnp.transpose(W, (1, 0, 2)).reshape(_D, _K * _DS)
    b2 = b.reshape(1, _K * _DS)
    A = jnp.tile(jnp.eye(_DS, dtype=jnp.float32), (_K, _K))
    grid = (_N // _TILE,)
    z, z2 = pl.pallas_call(
        _fused_body,
        grid=grid,
        in_specs=[
            pl.BlockSpec((_TILE, _D), lambda i: (i, 0)),
            pl.BlockSpec((_D, _K * _DS), lambda i: (0, 0)),
            pl.BlockSpec((1, _K * _DS), lambda i: (0, 0)),
            pl.BlockSpec((_K * _DS, _K * _DS), lambda i: (0, 0)),
        ],
        out_specs=[
            pl.BlockSpec((_TILE, _K * _DS), lambda i: (i, 0)),
            pl.BlockSpec((_TILE, _K * _DS), lambda i: (i, 0)),
        ],
        out_shape=[
            jax.ShapeDtypeStruct((_N, _K * _DS), jnp.float32),
            jax.ShapeDtypeStruct((_N, _K * _DS), jnp.float32),
        ],
        compiler_params=pltpu.CompilerParams(
            dimension_semantics=("parallel",),
        ),
    )(X, W2, b2, A)
    Z = z.reshape(_N, _K, _DS)
    _Z = z2.reshape(_N, 1, _K, _DS)
    return (Z, _Z)
